# baseline (speedup 1.0000x reference)
import functools

import jax
import jax.numpy as jnp
from jax import lax
from jax.experimental import pallas as pl
from jax.experimental.pallas import tpu as pltpu
from jax.experimental.pallas import tpu_sc as plsc

L = 200
B = 4096
H = 32
NC, NS = 2, 16
NW = NC * NS
Q = 512
UPL = B // Q
NU = (L * UPL) // NW
SUB = 128
NSUB = Q // SUB
D = 4

_mesh = plsc.VectorSubcoreMesh(core_axis_name="c", subcore_axis_name="s")


@functools.partial(
    pl.kernel,
    out_type=jax.ShapeDtypeStruct((L, B, H), jnp.float32),
    mesh=_mesh,
    compiler_params=pltpu.CompilerParams(use_tc_tiling_on_sc=False),
    scratch_types=[pltpu.VMEM((L, H), jnp.float32)]
    + [pltpu.VMEM((Q,), jnp.int32) for _ in range(D)]
    + [pltpu.VMEM((Q, H), jnp.float32) for _ in range(D)]
    + [pltpu.SemaphoreType.DMA for _ in range(3 * D)],
)
def _embed_kernel(in_hbm, emb_hbm, pos_hbm, out_hbm, pos_v, *refs):
    idxb = refs[:D]
    rows = refs[D:2 * D]
    isem = refs[2 * D:3 * D]
    gsem = refs[3 * D:4 * D]
    osem = refs[4 * D:]
    wid = lax.axis_index("s") * NC + lax.axis_index("c")
    u_base = wid * NU

    pltpu.sync_copy(pos_hbm, pos_v)

    def unit_lb(u):
        u_g = u_base + u
        l = u_g // UPL
        b0 = pl.multiple_of((u_g % UPL) * Q, Q)
        return l, b0

    def fire_idx(u):
        l, b0 = unit_lb(u)
        return pltpu.async_copy(
            in_hbm.at[l, pl.ds(b0, Q)], idxb[u % D], isem[u % D]
        )

    def fire_gather(u):
        b = u % D
        return [
            pltpu.async_copy(
                emb_hbm.at[idxb[b].at[pl.ds(j * SUB, SUB)]],
                rows[b].at[pl.ds(j * SUB, SUB)],
                gsem[b],
            )
            for j in range(NSUB)
        ]

    gdescs = {}
    odescs = {}
    idescs = {u: fire_idx(u) for u in range(D)}
    for u in range(D - 1):
        idescs.pop(u).wait()
        gdescs[u] = fire_gather(u)

    for u in range(NU):
        for d in gdescs.pop(u):
            d.wait()
        if u + D < NU:
            idescs[u + D] = fire_idx(u + D)
        v = u + D - 1
        if v < NU:
            if u >= 1:
                odescs.pop(u - 1).wait()
            idescs.pop(v).wait()
            gdescs[v] = fire_gather(v)
        b = u % D
        buf = rows[b]
        l, b0 = unit_lb(u)
        plo = pos_v[l, pl.ds(0, 16)]
        phi = pos_v[l, pl.ds(16, 16)]

        @plsc.parallel_loop(0, Q, 1, unroll=8)
        def _add(j):
            buf[j, pl.ds(0, 16)] = buf[j, pl.ds(0, 16)] + plo
            buf[j, pl.ds(16, 16)] = buf[j, pl.ds(16, 16)] + phi

        odescs[u] = pltpu.async_copy(
            buf, out_hbm.at[l, pl.ds(b0, Q)], osem[b]
        )

    for u in sorted(odescs):
        odescs.pop(u).wait()


def kernel(inputs, embedding, posembedding):
    return _embed_kernel(inputs, embedding, posembedding)


# speedup vs baseline: 1.0927x; 1.0927x over previous
import functools

import jax
import jax.numpy as jnp
from jax import lax
from jax.experimental import pallas as pl
from jax.experimental.pallas import tpu as pltpu
from jax.experimental.pallas import tpu_sc as plsc

L = 200
B = 4096
H = 32
V = 1000000
FLAT = L * B
NC, NS = 2, 16
NW = NC * NS

_mesh = plsc.VectorSubcoreMesh(core_axis_name="c", subcore_axis_name="s")

W = 512
NBLK_FULL = V // W
BLK_PER_W = NBLK_FULL // NW
TAIL0 = NBLK_FULL * W
PR = W // 4


@functools.partial(
    pl.kernel,
    out_type=jax.ShapeDtypeStruct((V // 4, 128), jnp.float32),
    mesh=_mesh,
    compiler_params=pltpu.CompilerParams(
        use_tc_tiling_on_sc=True, needs_layout_passes=False
    ),
    scratch_types=[pltpu.VMEM((H, W), jnp.float32) for _ in range(2)]
    + [pltpu.VMEM((PR, 128), jnp.float32) for _ in range(2)]
    + [pltpu.VMEM((64, H), jnp.float32)]
    + [pltpu.SemaphoreType.DMA for _ in range(4)],
)
def _detile_kernel(emb_t, tail_tab, p_hbm, a0, a1, pc0, pc1, tv, is0, is1, os0, os1):
    abufs, pcbufs = (a0, a1), (pc0, pc1)
    isems, osems = (is0, is1), (os0, os1)
    wid = lax.axis_index("s") * NC + lax.axis_index("c")
    blk0 = wid * BLK_PER_W
    hvec = lax.iota(jnp.int32, 16)
    row_lo = hvec
    row_hi = hvec + 16

    def fire_in(i):
        c0 = pl.multiple_of((blk0 + i) * W, W)
        return pltpu.async_copy(
            emb_t.at[:, pl.ds(c0, W)], abufs[i % 2], isems[i % 2]
        )

    def transpose_block(abuf, pcbuf, nrows):
        @plsc.parallel_loop(0, nrows, 1, unroll=1)
        def _t(r):
            for k in range(4):
                col = lax.broadcast(4 * r + k, (16,))
                pcbuf[r, pl.ds(k * 32, 16)] = plsc.load_gather(abuf, [row_lo, col])
                pcbuf[r, pl.ds(k * 32 + 16, 16)] = plsc.load_gather(abuf, [row_hi, col])

    idescs = {0: fire_in(0), 1: fire_in(1)}
    odescs = {}
    for i in range(BLK_PER_W):
        idescs.pop(i).wait()
        if i >= 2:
            odescs.pop(i - 2).wait()
        transpose_block(abufs[i % 2], pcbufs[i % 2], PR)
        odescs[i] = pltpu.async_copy(
            pcbufs[i % 2],
            p_hbm.at[pl.ds(pl.multiple_of((blk0 + i) * PR, 8), PR)],
            osems[i % 2],
        )
        if i + 2 < BLK_PER_W:
            idescs[i + 2] = fire_in(i + 2)
    for i in (BLK_PER_W - 2, BLK_PER_W - 1):
        odescs.pop(i).wait()

    @pl.when(wid == 0)
    def _extra():
        c0 = NW * BLK_PER_W * W
        pltpu.sync_copy(emb_t.at[:, pl.ds(c0, W)], a0)
        transpose_block(a0, pc0, PR)
        pltpu.sync_copy(pc0, p_hbm.at[pl.ds(NW * BLK_PER_W * PR, PR)])

    @pl.when(wid == NW - 1)
    def _tail():
        pltpu.sync_copy(tail_tab, tv)
        for r in range(16):
            for k in range(4):
                for m in range(2):
                    pc0[r, pl.ds(k * 32 + m * 16, 16)] = tv[4 * r + k, pl.ds(m * 16, 16)]
        pltpu.sync_copy(pc0.at[pl.ds(0, 16)], p_hbm.at[pl.ds(TAIL0 // 4, 16)])


Q = 512
UPL = B // Q
NU = FLAT // NW // Q
D = 3
OWORDS = 4 * (Q // 128) * 8 * 128


@functools.partial(
    pl.kernel,
    out_type=jax.ShapeDtypeStruct((L, 4, (B // 128) * 8 * 128), jnp.float32),
    mesh=_mesh,
    compiler_params=pltpu.CompilerParams(
        use_tc_tiling_on_sc=False, needs_layout_passes=False
    ),
    scratch_types=[pltpu.VMEM((H,), jnp.float32)]
    + [pltpu.VMEM((Q,), jnp.int32) for _ in range(D)]
    + [pltpu.VMEM((Q, H), jnp.float32) for _ in range(D)]
    + [pltpu.VMEM((OWORDS,), jnp.float32) for _ in range(D)]
    + [pltpu.SemaphoreType.DMA for _ in range(3 * D)],
)
def _gather_kernel(idx_hbm, tab_hbm, pos_hbm, out_hbm, pos_v, *refs):
    idxb = refs[:D]
    rows = refs[D:2 * D]
    obufs = refs[2 * D:3 * D]
    isem = refs[3 * D:4 * D]
    gsem = refs[4 * D:5 * D]
    osem = refs[5 * D:]
    wid = lax.axis_index("s") * NC + lax.axis_index("c")
    u_base = wid * NU
    hvec = lax.iota(jnp.int32, 16)
    gstride = (Q // 128) * 1024
    haddr_lo = (hvec // 8) * gstride + (hvec % 8) * 128
    haddr_hi = ((hvec + 16) // 8) * gstride + ((hvec + 16) % 8) * 128

    def fire_idx(u):
        off = pl.multiple_of((u_base + u) * Q, Q)
        return pltpu.async_copy(idx_hbm.at[pl.ds(off, Q)], idxb[u % D], isem[u % D])

    def fire_gather(u):
        b = u % D
        return [
            pltpu.async_copy(
                tab_hbm.at[idxb[b].at[pl.ds(j * 128, 128)]],
                rows[b].at[pl.ds(j * 128, 128)],
                gsem[b],
            )
            for j in range(Q // 128)
        ]

    gdescs = {}
    odescs = {}
    idescs = {u: fire_idx(u) for u in range(D)}
    for u in range(D - 1):
        idescs.pop(u).wait()
        gdescs[u] = fire_gather(u)

    for u in range(NU):
        for d in gdescs.pop(u):
            d.wait()
        if u + D < NU:
            idescs[u + D] = fire_idx(u + D)
        v = u + D - 1
        if v < NU:
            if u >= 1:
                for d in odescs.pop(u - 1):
                    d.wait()
            idescs.pop(v).wait()
            gdescs[v] = fire_gather(v)
        bi = u % D
        gbuf = rows[bi]
        obuf = obufs[bi]
        gu = u_base + u
        l = gu // UPL
        pltpu.sync_copy(
            pos_hbm.at[pl.ds(pl.multiple_of((gu // UPL) * H, 8), H)], pos_v
        )
        plo = pos_v[pl.ds(0, 16)]
        phi = pos_v[pl.ds(16, 16)]

        @plsc.parallel_loop(0, Q, 1, unroll=4)
        def _proc(j):
            sj = (j // 128) * 1024 + (j % 128)
            plsc.store_scatter(obuf, [haddr_lo + sj], gbuf[j, pl.ds(0, 16)] + plo)
            plsc.store_scatter(obuf, [haddr_hi + sj], gbuf[j, pl.ds(16, 16)] + phi)

        gw = (Q // 128) * 1024
        c0 = gu % UPL
        odescs[u] = [
            pltpu.async_copy(
                obuf.at[pl.ds(g * gw, gw)],
                out_hbm.at[l, g, pl.ds(pl.multiple_of(c0 * gw, 8), gw)],
                osem[bi],
            )
            for g in range(4)
        ]

    for u in sorted(odescs):
        for d in odescs.pop(u):
            d.wait()


def kernel(inputs, embedding, posembedding):
    p = _detile_kernel(embedding.T, embedding[TAIL0:])
    tab = p.reshape(V, H)
    idx1d = inputs.reshape(FLAT)
    posflat = posembedding.reshape(L * H)
    x = _gather_kernel(idx1d, tab, posflat)
    return (
        x.reshape(L, 4, B // 128, 8, 128)
        .transpose(0, 2, 4, 1, 3)
        .reshape(L, B, H)
    )


# speedup vs baseline: 1.0937x; 1.0009x over previous
import functools

import jax
import jax.numpy as jnp
from jax import lax
from jax.experimental import pallas as pl
from jax.experimental.pallas import tpu as pltpu
from jax.experimental.pallas import tpu_sc as plsc

L = 200
B = 4096
H = 32
V = 1000000
FLAT = L * B
NC, NS = 2, 16
NW = NC * NS

_mesh = plsc.VectorSubcoreMesh(core_axis_name="c", subcore_axis_name="s")

W = 512
NBLK_FULL = V // W
BLK_PER_W = NBLK_FULL // NW
TAIL0 = NBLK_FULL * W
PR = W // 4


@functools.partial(
    pl.kernel,
    out_type=jax.ShapeDtypeStruct((V // 4, 128), jnp.float32),
    mesh=_mesh,
    compiler_params=pltpu.CompilerParams(
        use_tc_tiling_on_sc=True, needs_layout_passes=False
    ),
    scratch_types=[pltpu.VMEM((H, W), jnp.float32) for _ in range(2)]
    + [pltpu.VMEM((PR, 128), jnp.float32) for _ in range(2)]
    + [pltpu.VMEM((64, H), jnp.float32)]
    + [pltpu.SemaphoreType.DMA for _ in range(4)],
)
def _detile_kernel(emb_t, tail_tab, p_hbm, a0, a1, pc0, pc1, tv, is0, is1, os0, os1):
    abufs, pcbufs = (a0, a1), (pc0, pc1)
    isems, osems = (is0, is1), (os0, os1)
    wid = lax.axis_index("s") * NC + lax.axis_index("c")
    blk0 = wid * BLK_PER_W
    hvec = lax.iota(jnp.int32, 16)
    row_lo = hvec
    row_hi = hvec + 16

    def in_slice(i):
        return emb_t.at[:, pl.ds(pl.multiple_of((blk0 + i) * W, W), W)]

    def out_slice(i):
        return p_hbm.at[pl.ds(pl.multiple_of((blk0 + i) * PR, 8), PR)]

    def transpose_block(abuf, pcbuf, nrows):
        @plsc.parallel_loop(0, nrows, 1, unroll=4)
        def _t(r):
            for k in range(4):
                col = lax.broadcast(4 * r + k, (16,))
                pcbuf[r, pl.ds(k * 32, 16)] = plsc.load_gather(abuf, [row_lo, col])
                pcbuf[r, pl.ds(k * 32 + 16, 16)] = plsc.load_gather(abuf, [row_hi, col])

    pltpu.async_copy(in_slice(0), a0, is0)
    pltpu.async_copy(in_slice(1), a1, is1)

    def pair_body(g, carry):
        for b, abuf, pcbuf, isem, osem in ((0, a0, pc0, is0, os0), (1, a1, pc1, is1, os1)):
            i = 2 * g + b
            pltpu.make_async_copy(in_slice(i), abuf, isem).wait()

            @pl.when(g >= 1)
            def _drain():
                pltpu.make_async_copy(pcbuf, out_slice(i - 2), osem).wait()

            transpose_block(abuf, pcbuf, PR)
            pltpu.async_copy(pcbuf, out_slice(i), osem)

            @pl.when(i + 2 < BLK_PER_W)
            def _prefetch():
                pltpu.async_copy(in_slice(i + 2), abuf, isem)
        return carry

    lax.fori_loop(0, BLK_PER_W // 2, pair_body, 0)
    last = BLK_PER_W - 1
    pltpu.make_async_copy(in_slice(last), a0, is0).wait()
    pltpu.make_async_copy(pc0, out_slice(last - 2), os0).wait()
    transpose_block(a0, pc0, PR)
    pltpu.async_copy(pc0, out_slice(last), os0)
    pltpu.make_async_copy(pc1, out_slice(last - 1), os1).wait()
    pltpu.make_async_copy(pc0, out_slice(last), os0).wait()

    @pl.when(wid == 0)
    def _extra():
        c0 = NW * BLK_PER_W * W
        pltpu.sync_copy(emb_t.at[:, pl.ds(c0, W)], a0)
        transpose_block(a0, pc0, PR)
        pltpu.sync_copy(pc0, p_hbm.at[pl.ds(NW * BLK_PER_W * PR, PR)])

    @pl.when(wid == NW - 1)
    def _tail():
        pltpu.sync_copy(tail_tab, tv)
        for r in range(16):
            for k in range(4):
                for m in range(2):
                    pc0[r, pl.ds(k * 32 + m * 16, 16)] = tv[4 * r + k, pl.ds(m * 16, 16)]
        pltpu.sync_copy(pc0.at[pl.ds(0, 16)], p_hbm.at[pl.ds(TAIL0 // 4, 16)])


Q = 512
UPL = B // Q
NU = FLAT // NW // Q
D = 3
OWORDS = 4 * (Q // 128) * 8 * 128


@functools.partial(
    pl.kernel,
    out_type=jax.ShapeDtypeStruct((L, 4, (B // 128) * 8 * 128), jnp.float32),
    mesh=_mesh,
    compiler_params=pltpu.CompilerParams(
        use_tc_tiling_on_sc=False, needs_layout_passes=False
    ),
    scratch_types=[pltpu.VMEM((H,), jnp.float32)]
    + [pltpu.VMEM((Q,), jnp.int32) for _ in range(D)]
    + [pltpu.VMEM((Q, H), jnp.float32) for _ in range(D)]
    + [pltpu.VMEM((OWORDS,), jnp.float32) for _ in range(D)]
    + [pltpu.SemaphoreType.DMA for _ in range(3 * D)],
)
def _gather_kernel(idx_hbm, tab_hbm, pos_hbm, out_hbm, pos_v, *refs):
    idxb = refs[:D]
    rows = refs[D:2 * D]
    obufs = refs[2 * D:3 * D]
    isem = refs[3 * D:4 * D]
    gsem = refs[4 * D:5 * D]
    osem = refs[5 * D:]
    wid = lax.axis_index("s") * NC + lax.axis_index("c")
    u_base = wid * NU
    hvec = lax.iota(jnp.int32, 16)
    gstride = (Q // 128) * 1024
    haddr_lo = (hvec // 8) * gstride + (hvec % 8) * 128
    haddr_hi = ((hvec + 16) // 8) * gstride + ((hvec + 16) % 8) * 128

    def fire_idx(u):
        off = pl.multiple_of((u_base + u) * Q, Q)
        return pltpu.async_copy(idx_hbm.at[pl.ds(off, Q)], idxb[u % D], isem[u % D])

    def fire_gather(u):
        b = u % D
        return [
            pltpu.async_copy(
                tab_hbm.at[idxb[b].at[pl.ds(j * 128, 128)]],
                rows[b].at[pl.ds(j * 128, 128)],
                gsem[b],
            )
            for j in range(Q // 128)
        ]

    gdescs = {}
    odescs = {}
    idescs = {u: fire_idx(u) for u in range(D)}
    for u in range(D - 1):
        idescs.pop(u).wait()
        gdescs[u] = fire_gather(u)

    for u in range(NU):
        for d in gdescs.pop(u):
            d.wait()
        if u + D < NU:
            idescs[u + D] = fire_idx(u + D)
        v = u + D - 1
        if v < NU:
            if u >= 1:
                for d in odescs.pop(u - 1):
                    d.wait()
            idescs.pop(v).wait()
            gdescs[v] = fire_gather(v)
        bi = u % D
        gbuf = rows[bi]
        obuf = obufs[bi]
        gu = u_base + u
        l = gu // UPL
        pltpu.sync_copy(
            pos_hbm.at[pl.ds(pl.multiple_of((gu // UPL) * H, 8), H)], pos_v
        )
        plo = pos_v[pl.ds(0, 16)]
        phi = pos_v[pl.ds(16, 16)]

        @plsc.parallel_loop(0, Q, 1, unroll=2)
        def _proc(j):
            sj = lax.shift_left(lax.shift_right_logical(j, 7), 10) + lax.bitwise_and(j, 127)
            plsc.store_scatter(obuf, [haddr_lo + sj], gbuf[j, pl.ds(0, 16)] + plo)
            plsc.store_scatter(obuf, [haddr_hi + sj], gbuf[j, pl.ds(16, 16)] + phi)

        gw = (Q // 128) * 1024
        c0 = gu % UPL
        odescs[u] = [
            pltpu.async_copy(
                obuf.at[pl.ds(g * gw, gw)],
                out_hbm.at[l, g, pl.ds(pl.multiple_of(c0 * gw, 8), gw)],
                osem[bi],
            )
            for g in range(4)
        ]

    for u in sorted(odescs):
        for d in odescs.pop(u):
            d.wait()


def kernel(inputs, embedding, posembedding):
    p = _detile_kernel(embedding.T, embedding[TAIL0:])
    tab = p.reshape(V, H)
    idx1d = inputs.reshape(FLAT)
    posflat = posembedding.reshape(L * H)
    x = _gather_kernel(idx1d, tab, posflat)
    return (
        x.reshape(L, 4, B // 128, 8, 128)
        .transpose(0, 2, 4, 1, 3)
        .reshape(L, B, H)
    )


# speedup vs baseline: 1.1104x; 1.0152x over previous
import functools

import jax
import jax.numpy as jnp
from jax import lax
from jax.experimental import pallas as pl
from jax.experimental.pallas import tpu as pltpu
from jax.experimental.pallas import tpu_sc as plsc

L = 200
B = 4096
H = 32
V = 1000000
FLAT = L * B
NC, NS = 2, 16
NW = NC * NS

_mesh = plsc.VectorSubcoreMesh(core_axis_name="c", subcore_axis_name="s")

W = 512
NBLK_FULL = V // W
BLK_PER_W = NBLK_FULL // NW
TAIL0 = NBLK_FULL * W
PR = W // 4


@functools.partial(
    pl.kernel,
    out_type=jax.ShapeDtypeStruct((V // 4, 128), jnp.float32),
    mesh=_mesh,
    compiler_params=pltpu.CompilerParams(
        use_tc_tiling_on_sc=True, needs_layout_passes=False
    ),
    scratch_types=[pltpu.VMEM((H, W), jnp.float32) for _ in range(2)]
    + [pltpu.VMEM((PR, 128), jnp.float32) for _ in range(2)]
    + [pltpu.VMEM((64, H), jnp.float32)]
    + [pltpu.SemaphoreType.DMA for _ in range(4)],
)
def _detile_kernel(emb_t, tail_tab, p_hbm, a0, a1, pc0, pc1, tv, is0, is1, os0, os1):
    abufs, pcbufs = (a0, a1), (pc0, pc1)
    isems, osems = (is0, is1), (os0, os1)
    wid = lax.axis_index("s") * NC + lax.axis_index("c")
    blk0 = wid * BLK_PER_W
    hvec = lax.iota(jnp.int32, 16)
    row_lo = hvec
    row_hi = hvec + 16

    def in_slice(i):
        return emb_t.at[:, pl.ds(pl.multiple_of((blk0 + i) * W, W), W)]

    def out_slice(i):
        return p_hbm.at[pl.ds(pl.multiple_of((blk0 + i) * PR, 8), PR)]

    def transpose_block(abuf, pcbuf, nrows):
        @plsc.parallel_loop(0, nrows, 1, unroll=4)
        def _t(r):
            for k in range(4):
                col = lax.broadcast(4 * r + k, (16,))
                pcbuf[r, pl.ds(k * 32, 16)] = plsc.load_gather(abuf, [row_lo, col])
                pcbuf[r, pl.ds(k * 32 + 16, 16)] = plsc.load_gather(abuf, [row_hi, col])

    pltpu.async_copy(in_slice(0), a0, is0)
    pltpu.async_copy(in_slice(1), a1, is1)

    def pair_body(g, carry):
        for b, abuf, pcbuf, isem, osem in ((0, a0, pc0, is0, os0), (1, a1, pc1, is1, os1)):
            i = 2 * g + b
            pltpu.make_async_copy(in_slice(i), abuf, isem).wait()

            @pl.when(g >= 1)
            def _drain():
                pltpu.make_async_copy(pcbuf, out_slice(i - 2), osem).wait()

            transpose_block(abuf, pcbuf, PR)
            pltpu.async_copy(pcbuf, out_slice(i), osem)

            @pl.when(i + 2 < BLK_PER_W)
            def _prefetch():
                pltpu.async_copy(in_slice(i + 2), abuf, isem)
        return carry

    lax.fori_loop(0, BLK_PER_W // 2, pair_body, 0)
    last = BLK_PER_W - 1
    pltpu.make_async_copy(in_slice(last), a0, is0).wait()
    pltpu.make_async_copy(pc0, out_slice(last - 2), os0).wait()
    transpose_block(a0, pc0, PR)
    pltpu.async_copy(pc0, out_slice(last), os0)
    pltpu.make_async_copy(pc1, out_slice(last - 1), os1).wait()
    pltpu.make_async_copy(pc0, out_slice(last), os0).wait()

    @pl.when(wid == 0)
    def _extra():
        c0 = NW * BLK_PER_W * W
        pltpu.sync_copy(emb_t.at[:, pl.ds(c0, W)], a0)
        transpose_block(a0, pc0, PR)
        pltpu.sync_copy(pc0, p_hbm.at[pl.ds(NW * BLK_PER_W * PR, PR)])

    @pl.when(wid == NW - 1)
    def _tail():
        pltpu.sync_copy(tail_tab, tv)
        for r in range(16):
            for k in range(4):
                for m in range(2):
                    pc0[r, pl.ds(k * 32 + m * 16, 16)] = tv[4 * r + k, pl.ds(m * 16, 16)]
        pltpu.sync_copy(pc0.at[pl.ds(0, 16)], p_hbm.at[pl.ds(TAIL0 // 4, 16)])


Q = 512
UPL = B // Q
NU = FLAT // NW // Q
D = 3
OWORDS = 4 * (Q // 128) * 8 * 128


@functools.partial(
    pl.kernel,
    out_type=jax.ShapeDtypeStruct((L, 4, (B // 128) * 8 * 128), jnp.float32),
    mesh=_mesh,
    compiler_params=pltpu.CompilerParams(
        use_tc_tiling_on_sc=False, needs_layout_passes=False
    ),
    scratch_types=[pltpu.VMEM((H,), jnp.float32)]
    + [pltpu.VMEM((Q,), jnp.int32) for _ in range(D)]
    + [pltpu.VMEM((Q, H), jnp.float32) for _ in range(D)]
    + [pltpu.VMEM((OWORDS,), jnp.float32) for _ in range(D)]
    + [pltpu.SemaphoreType.DMA for _ in range(3 * D)],
)
def _gather_kernel(idx_hbm, tab_hbm, pos_hbm, out_hbm, pos_v, *refs):
    idxb = refs[:D]
    rows = refs[D:2 * D]
    obufs = refs[2 * D:3 * D]
    isem = refs[3 * D:4 * D]
    gsem = refs[4 * D:5 * D]
    osem = refs[5 * D:]
    wid = lax.axis_index("s") * NC + lax.axis_index("c")
    u_base = wid * NU
    hvec = lax.iota(jnp.int32, 16)
    gstride = (Q // 128) * 1024
    haddr_lo = (hvec // 8) * gstride + (hvec % 8) * 128
    haddr_hi = ((hvec + 16) // 8) * gstride + ((hvec + 16) % 8) * 128

    def fire_idx(u):
        off = pl.multiple_of((u_base + u) * Q, Q)
        return pltpu.async_copy(idx_hbm.at[pl.ds(off, Q)], idxb[u % D], isem[u % D])

    def fire_gather(u):
        b = u % D
        return [
            pltpu.async_copy(
                tab_hbm.at[idxb[b].at[pl.ds(j * 128, 128)]],
                rows[b].at[pl.ds(j * 128, 128)],
                gsem[b],
            )
            for j in range(Q // 128)
        ]

    gdescs = {}
    odescs = {}
    idescs = {u: fire_idx(u) for u in range(D)}
    for u in range(D - 1):
        idescs.pop(u).wait()
        gdescs[u] = fire_gather(u)

    for u in range(NU):
        for d in gdescs.pop(u):
            d.wait()
        if u + D < NU:
            idescs[u + D] = fire_idx(u + D)
        v = u + D - 1
        if v < NU:
            if u >= 1:
                for d in odescs.pop(u - 1):
                    d.wait()
            idescs.pop(v).wait()
            gdescs[v] = fire_gather(v)
        bi = u % D
        gbuf = rows[bi]
        obuf = obufs[bi]
        gu = u_base + u
        l = gu // UPL
        pltpu.sync_copy(
            pos_hbm.at[pl.ds(pl.multiple_of((gu // UPL) * H, 8), H)], pos_v
        )
        plo = pos_v[pl.ds(0, 16)]
        phi = pos_v[pl.ds(16, 16)]

        @plsc.parallel_loop(0, Q, 1, unroll=4)
        def _proc(j):
            sj = lax.shift_left(lax.shift_right_logical(j, 7), 10) + lax.bitwise_and(j, 127)
            plsc.store_scatter(obuf, [haddr_lo + sj], gbuf[j, pl.ds(0, 16)] + plo)
            plsc.store_scatter(obuf, [haddr_hi + sj], gbuf[j, pl.ds(16, 16)] + phi)

        gw = (Q // 128) * 1024
        c0 = gu % UPL
        odescs[u] = [
            pltpu.async_copy(
                obuf.at[pl.ds(g * gw, gw)],
                out_hbm.at[l, g, pl.ds(pl.multiple_of(c0 * gw, 8), gw)],
                osem[bi],
            )
            for g in range(4)
        ]

    for u in sorted(odescs):
        for d in odescs.pop(u):
            d.wait()


def kernel(inputs, embedding, posembedding):
    p = _detile_kernel(embedding.T, embedding[TAIL0:])
    tab = p.reshape(V, H)
    idx1d = inputs.reshape(FLAT)
    posflat = posembedding.reshape(L * H)
    x = _gather_kernel(idx1d, tab, posflat)
    return (
        x.reshape(L, 4, B // 128, 8, 128)
        .transpose(0, 2, 4, 1, 3)
        .reshape(L, B, H)
    )
